# baseline (speedup 1.0000x reference)
import functools

import jax
import jax.numpy as jnp
from jax import lax
from jax.experimental import pallas as pl
from jax.experimental.pallas import tpu as pltpu
from jax.experimental.pallas import tpu_sc as plsc

DIM = 64
NUM_ITEM = 1000
NI_PAD = 1024
BATCH = 1024

_NC, _NS = 2, 16
_NW = _NC * _NS


def _make_sc_gather(n_rows: int):
    b_per_w = n_rows // _NW
    mesh = plsc.VectorSubcoreMesh(core_axis_name="c", subcore_axis_name="s")

    @functools.partial(
        pl.kernel,
        mesh=mesh,
        out_type=jax.ShapeDtypeStruct((n_rows, DIM), jnp.float32),
        scratch_types=[
            pltpu.VMEM((b_per_w,), jnp.int32),
            pltpu.VMEM((b_per_w, 8, DIM), jnp.float32),
            pltpu.VMEM((b_per_w, DIM), jnp.float32),
            pltpu.SemaphoreType.DMA,
        ],
    )
    def gk(idx_hbm, out_hbm, grp_v, rows_v, out_v, sem):
        wid = lax.axis_index("s") * _NC + lax.axis_index("c")
        base = wid * b_per_w
        pltpu.sync_copy(idx_hbm.at[pl.ds(base, b_per_w)], grp_v)
        pltpu.sync_copy(out_v, out_hbm.at[pl.ds(base, b_per_w)])

    return gk


_gather_cache = {}


def _sc_gather(idx):
    n = idx.shape[0]
    if n not in _gather_cache:
        _gather_cache[n] = _make_sc_gather(n)
    return _gather_cache[n](idx)


_BB = 128
_IT = 256


def _dense_body(table_ref, u_ref, it_ref, at_ref, buy_ref, rank_ref, pred_ref,
                tbuf, tsem):
    @pl.when(jnp.logical_and(pl.program_id(0) == 0, pl.program_id(1) == 0))
    def _():
        pltpu.make_async_copy(table_ref.at[pl.ds(0, 8)], tbuf, tsem).start()
        pltpu.make_async_copy(table_ref.at[pl.ds(0, 8)], tbuf, tsem).wait()
    j = pl.program_id(1)
    u = u_ref[...]
    at = at_ref[...]
    buy = buy_ref[0:1, :]
    up = u + buy

    acc = jnp.zeros((_BB, _IT), jnp.float32)
    for d in range(DIM):
        col = up[:, d:d + 1]
        row = at[d:d + 1, :]
        acc = acc + jnp.abs(col - row)

    scores = jnp.dot(u, at, preferred_element_type=jnp.float32)
    rank_ref[...] = jax.nn.sigmoid(acc + scores)

    @pl.when(j == 0)
    def _():
        it = it_ref[...]
        ps = jnp.sum(u * it, axis=1)
        pd = jnp.sum(jnp.abs(up - it), axis=1)
        pred_ref[...] = jax.nn.sigmoid(pd + ps)


def _dense(entity_table, user_embed, item_embed, a_t, buy8):
    grid = (BATCH // _BB, NI_PAD // _IT)
    rank, pred = pl.pallas_call(
        _dense_body,
        grid=grid,
        in_specs=[
            pl.BlockSpec(memory_space=pltpu.HBM),
            pl.BlockSpec((_BB, DIM), lambda i, j: (i, 0)),
            pl.BlockSpec((_BB, DIM), lambda i, j: (i, 0)),
            pl.BlockSpec((DIM, _IT), lambda i, j: (0, j)),
            pl.BlockSpec((8, DIM), lambda i, j: (0, 0)),
        ],
        out_specs=[
            pl.BlockSpec((_BB, _IT), lambda i, j: (i, j)),
            pl.BlockSpec((_BB,), lambda i, j: (i,)),
        ],
        out_shape=[
            jax.ShapeDtypeStruct((BATCH, NI_PAD), jnp.float32),
            jax.ShapeDtypeStruct((BATCH,), jnp.float32),
        ],
        scratch_shapes=[
            pltpu.VMEM((8, DIM), jnp.float32),
            pltpu.SemaphoreType.DMA,
        ],
    )(entity_table, user_embed, item_embed, a_t, buy8)
    return rank, pred


def kernel(users, items, entity_table, relation_table):
    users = users.astype(jnp.int32)
    items = items.astype(jnp.int32)
    idx = jnp.concatenate([users, items])
    gathered = _sc_gather(idx)
    user_embed = gathered[:BATCH]
    item_embed = gathered[BATCH:]

    all_items = entity_table[:NUM_ITEM]
    a_t = jnp.zeros((DIM, NI_PAD), jnp.float32).at[:, :NUM_ITEM].set(all_items.T)
    buy8 = jnp.broadcast_to(relation_table[-1], (8, DIM))

    rank, pred = _dense(entity_table, user_embed, item_embed, a_t, buy8)
    return (pred, rank[:, :NUM_ITEM])


# speedup vs baseline: 4.5817x; 4.5817x over previous
import jax
import jax.numpy as jnp
from jax import lax
from jax.experimental import pallas as pl
from jax.experimental.pallas import tpu as pltpu

DIM = 64
NUM_ITEM = 1000
NI_PAD = 1024
BATCH = 1024

_BB = 128
_IT = 256


def _gather_body(wcol_sref, clane_ref, table_ref, out_ref, wbuf, wsem):
    i = pl.program_id(0)
    copies = []
    for b in range(_BB):
        col0 = pl.multiple_of(wcol_sref[i * _BB + b], 128)
        copies.append(
            pltpu.make_async_copy(
                table_ref.at[:, pl.ds(col0, 128)], wbuf.at[b], wsem
            )
        )
    for c in copies:
        c.start()
    for c in copies:
        c.wait()
    w = wbuf[...]
    c_b = clane_ref[...]
    lane = lax.broadcasted_iota(jnp.int32, (_BB, DIM, 128), 2)
    mask = lane == c_b[:, None, None]
    out_ref[...] = jnp.where(mask, w, 0.0).sum(axis=2)


def _gather(table_t, wcol, clane):
    return pl.pallas_call(
        _gather_body,
        grid_spec=pltpu.PrefetchScalarGridSpec(
            num_scalar_prefetch=1,
            grid=(BATCH // _BB,),
            in_specs=[
                pl.BlockSpec((_BB,), lambda i, s: (i,)),
                pl.BlockSpec(memory_space=pltpu.HBM),
            ],
            out_specs=pl.BlockSpec((_BB, DIM), lambda i, s: (i, 0)),
            scratch_shapes=[
                pltpu.VMEM((_BB, DIM, 128), jnp.float32),
                pltpu.SemaphoreType.DMA,
            ],
        ),
        out_shape=jax.ShapeDtypeStruct((BATCH, DIM), jnp.float32),
    )(wcol, clane, table_t)


def _dense_body(u_ref, items_ref, atf_ref, at_ref, buy_ref, rank_ref, pred_ref):
    j = pl.program_id(1)
    u = u_ref[...]
    at = at_ref[...]
    buy = buy_ref[0:1, :]
    up = u + buy

    acc = jnp.zeros((_BB, _IT), jnp.float32)
    for d in range(DIM):
        col = up[:, d:d + 1]
        row = at[d:d + 1, :]
        acc = acc + jnp.abs(col - row)

    scores = jnp.dot(u, at, preferred_element_type=jnp.float32)
    rank_ref[...] = jax.nn.sigmoid(acc + scores)

    @pl.when(j == 0)
    def _():
        atf = atf_ref[...]
        items = items_ref[...]
        cols = lax.broadcasted_iota(jnp.int32, (_BB, NI_PAD), 1)
        onehot = (cols == items[:, None]).astype(jnp.float32)
        ie = lax.dot_general(
            onehot, atf, (((1,), (1,)), ((), ())),
            preferred_element_type=jnp.float32,
        )
        ps = jnp.sum(u * ie, axis=1)
        pd = jnp.sum(jnp.abs(up - ie), axis=1)
        pred_ref[...] = jax.nn.sigmoid(pd + ps)


def _dense(user_embed, items, a_t, buy8):
    grid = (BATCH // _BB, NI_PAD // _IT)
    rank, pred = pl.pallas_call(
        _dense_body,
        grid=grid,
        in_specs=[
            pl.BlockSpec((_BB, DIM), lambda i, j: (i, 0)),
            pl.BlockSpec((_BB,), lambda i, j: (i,)),
            pl.BlockSpec((DIM, NI_PAD), lambda i, j: (0, 0)),
            pl.BlockSpec((DIM, _IT), lambda i, j: (0, j)),
            pl.BlockSpec((8, DIM), lambda i, j: (0, 0)),
        ],
        out_specs=[
            pl.BlockSpec((_BB, _IT), lambda i, j: (i, j)),
            pl.BlockSpec((_BB,), lambda i, j: (i,)),
        ],
        out_shape=[
            jax.ShapeDtypeStruct((BATCH, NI_PAD), jnp.float32),
            jax.ShapeDtypeStruct((BATCH,), jnp.float32),
        ],
    )(user_embed, items, a_t, a_t, buy8)
    return rank, pred


def kernel(users, items, entity_table, relation_table):
    users = users.astype(jnp.int32)
    items = items.astype(jnp.int32)
    table_t = entity_table.T
    wcol = (users >> 7) << 7
    clane = users & 127
    user_embed = _gather(table_t, wcol, clane)

    a_t = jnp.pad(table_t[:, :NUM_ITEM], ((0, 0), (0, NI_PAD - NUM_ITEM)))
    buy8 = jnp.broadcast_to(relation_table[-1], (8, DIM))

    rank, pred = _dense(user_embed, items, a_t, buy8)
    return (pred, rank[:, :NUM_ITEM])


# speedup vs baseline: 5.2989x; 1.1565x over previous
import jax
import jax.numpy as jnp
from jax import lax
from jax.experimental import pallas as pl
from jax.experimental.pallas import tpu as pltpu

DIM = 64
NUM_ITEM = 1000
NI_PAD = 1024
BATCH = 1024

_BB = 128
_IT = 256


def _gather_body(wcol_sref, clane_ref, table_ref, out_ref, wbuf, wsem):
    i = pl.program_id(0)
    n = pl.num_programs(0)

    def fire(tile, slot):
        for b in range(_BB):
            col0 = pl.multiple_of(wcol_sref[tile * _BB + b], 128)
            pltpu.make_async_copy(
                table_ref.at[:, pl.ds(col0, 128)], wbuf.at[slot, b],
                wsem.at[slot],
            ).start()

    @pl.when(i == 0)
    def _():
        fire(0, 0)

    @pl.when(i + 1 < n)
    def _():
        fire(i + 1, (i + 1) % 2)

    sl = i % 2
    for b in range(_BB):
        pltpu.make_async_copy(
            table_ref.at[:, pl.ds(0, 128)], wbuf.at[sl, b], wsem.at[sl]
        ).wait()
    w = wbuf[sl]
    c_b = clane_ref[...]
    lane = lax.broadcasted_iota(jnp.int32, (_BB, DIM, 128), 2)
    mask = lane == c_b[:, None, None]
    out_ref[...] = jnp.where(mask, w, 0.0).sum(axis=2)


def _gather(table_t, wcol, clane):
    return pl.pallas_call(
        _gather_body,
        grid_spec=pltpu.PrefetchScalarGridSpec(
            num_scalar_prefetch=1,
            grid=(BATCH // _BB,),
            in_specs=[
                pl.BlockSpec((_BB,), lambda i, s: (i,)),
                pl.BlockSpec(memory_space=pltpu.HBM),
            ],
            out_specs=pl.BlockSpec((_BB, DIM), lambda i, s: (i, 0)),
            scratch_shapes=[
                pltpu.VMEM((2, _BB, DIM, 128), jnp.float32),
                pltpu.SemaphoreType.DMA((2,)),
            ],
        ),
        out_shape=jax.ShapeDtypeStruct((BATCH, DIM), jnp.float32),
    )(wcol, clane, table_t)


def _dense_body(u_ref, items_ref, atf_ref, at_ref, buy_ref, rank_ref, pred_ref):
    j = pl.program_id(1)
    u = u_ref[...]
    at = at_ref[...]
    buy = buy_ref[0:1, :]
    up = u + buy

    acc = jnp.zeros((_BB, _IT), jnp.float32)
    for d in range(DIM):
        col = up[:, d:d + 1]
        row = at[d:d + 1, :]
        acc = acc + jnp.abs(col - row)

    scores = jnp.dot(u, at, preferred_element_type=jnp.float32)
    rank_ref[...] = jax.nn.sigmoid(acc + scores)

    @pl.when(j == 0)
    def _():
        atf = atf_ref[...]
        items = items_ref[...]
        cols = lax.broadcasted_iota(jnp.int32, (_BB, NI_PAD), 1)
        onehot = (cols == items[:, None]).astype(jnp.float32)
        ie = lax.dot_general(
            onehot, atf, (((1,), (1,)), ((), ())),
            preferred_element_type=jnp.float32,
        )
        ps = jnp.sum(u * ie, axis=1)
        pd = jnp.sum(jnp.abs(up - ie), axis=1)
        pred_ref[...] = jax.nn.sigmoid(pd + ps)


def _dense(user_embed, items, a_t, buy8):
    grid = (BATCH // _BB, NI_PAD // _IT)
    rank, pred = pl.pallas_call(
        _dense_body,
        grid=grid,
        in_specs=[
            pl.BlockSpec((_BB, DIM), lambda i, j: (i, 0)),
            pl.BlockSpec((_BB,), lambda i, j: (i,)),
            pl.BlockSpec((DIM, NI_PAD), lambda i, j: (0, 0)),
            pl.BlockSpec((DIM, _IT), lambda i, j: (0, j)),
            pl.BlockSpec((8, DIM), lambda i, j: (0, 0)),
        ],
        out_specs=[
            pl.BlockSpec((_BB, _IT), lambda i, j: (i, j)),
            pl.BlockSpec((_BB,), lambda i, j: (i,)),
        ],
        out_shape=[
            jax.ShapeDtypeStruct((BATCH, NUM_ITEM), jnp.float32),
            jax.ShapeDtypeStruct((BATCH,), jnp.float32),
        ],
    )(user_embed, items, a_t, a_t, buy8)
    return rank, pred


def kernel(users, items, entity_table, relation_table):
    users = users.astype(jnp.int32)
    items = items.astype(jnp.int32)
    table_t = entity_table.T
    wcol = (users >> 7) << 7
    clane = users & 127
    user_embed = _gather(table_t, wcol, clane)

    a_t = table_t[:, :NI_PAD]
    buy8 = jnp.broadcast_to(relation_table[-1], (8, DIM))

    rank, pred = _dense(user_embed, items, a_t, buy8)
    return (pred, rank)


# speedup vs baseline: 5.4959x; 1.0372x over previous
import jax
import jax.numpy as jnp
from jax import lax
from jax.experimental import pallas as pl
from jax.experimental.pallas import tpu as pltpu

DIM = 64
NUM_ITEM = 1000
NI_PAD = 1024
BATCH = 1024

_BB = 128
_IT = 256
_NJ = NI_PAD // _IT


def _body(wcol_sref, clane_ref, items_ref, atf_ref, at_ref, buy_ref,
          table_ref, rank_ref, pred_ref, wbuf, wsem, u_scr):
    i = pl.program_id(0)
    j = pl.program_id(1)
    n = pl.num_programs(0)

    def fire(tile, slot):
        for b in range(_BB):
            col0 = pl.multiple_of(wcol_sref[tile * _BB + b], 128)
            pltpu.make_async_copy(
                table_ref.at[:, pl.ds(col0, 128)], wbuf.at[slot, b],
                wsem.at[slot],
            ).start()

    @pl.when(j == 0)
    def _():
        @pl.when(i == 0)
        def _():
            fire(0, 0)

        @pl.when(i + 1 < n)
        def _():
            fire(i + 1, (i + 1) % 2)

        sl = i % 2
        for b in range(_BB):
            pltpu.make_async_copy(
                table_ref.at[:, pl.ds(0, 128)], wbuf.at[sl, b], wsem.at[sl]
            ).wait()
        w = wbuf[sl]
        c_b = clane_ref[...]
        lane = lax.broadcasted_iota(jnp.int32, (_BB, DIM, 128), 2)
        mask = lane == c_b[:, None, None]
        u_scr[...] = jnp.where(mask, w, 0.0).sum(axis=2)

    u = u_scr[...]
    at = at_ref[...]
    buy = buy_ref[0:1, :]
    up = u + buy

    acc = jnp.zeros((_BB, _IT), jnp.float32)
    for d in range(DIM):
        col = up[:, d:d + 1]
        row = at[d:d + 1, :]
        acc = acc + jnp.abs(col - row)

    scores = jnp.dot(u, at, preferred_element_type=jnp.float32)
    rank_ref[...] = jax.nn.sigmoid(acc + scores)

    @pl.when(j == 0)
    def _():
        atf = atf_ref[...]
        items = items_ref[...]
        cols = lax.broadcasted_iota(jnp.int32, (_BB, NI_PAD), 1)
        onehot = (cols == items[:, None]).astype(jnp.float32)
        ie = lax.dot_general(
            onehot, atf, (((1,), (1,)), ((), ())),
            preferred_element_type=jnp.float32,
        )
        ps = jnp.sum(u * ie, axis=1)
        pd = jnp.sum(jnp.abs(up - ie), axis=1)
        pred_ref[...] = jax.nn.sigmoid(pd + ps)


def _fused(table_t, wcol, clane, items, a_t, buy8):
    rank, pred = pl.pallas_call(
        _body,
        grid_spec=pltpu.PrefetchScalarGridSpec(
            num_scalar_prefetch=1,
            grid=(BATCH // _BB, _NJ),
            in_specs=[
                pl.BlockSpec((_BB,), lambda i, j, s: (i,)),
                pl.BlockSpec((_BB,), lambda i, j, s: (i,)),
                pl.BlockSpec((DIM, NI_PAD), lambda i, j, s: (0, 0)),
                pl.BlockSpec((DIM, _IT), lambda i, j, s: (0, j)),
                pl.BlockSpec((8, DIM), lambda i, j, s: (0, 0)),
                pl.BlockSpec(memory_space=pltpu.HBM),
            ],
            out_specs=[
                pl.BlockSpec((_BB, _IT), lambda i, j, s: (i, j)),
                pl.BlockSpec((_BB,), lambda i, j, s: (i,)),
            ],
            scratch_shapes=[
                pltpu.VMEM((2, _BB, DIM, 128), jnp.float32),
                pltpu.SemaphoreType.DMA((2,)),
                pltpu.VMEM((_BB, DIM), jnp.float32),
            ],
        ),
        out_shape=[
            jax.ShapeDtypeStruct((BATCH, NUM_ITEM), jnp.float32),
            jax.ShapeDtypeStruct((BATCH,), jnp.float32),
        ],
    )(wcol, clane, items, a_t, a_t, buy8, table_t)
    return rank, pred


def kernel(users, items, entity_table, relation_table):
    users = users.astype(jnp.int32)
    items = items.astype(jnp.int32)
    table_t = entity_table.T
    wcol = (users >> 7) << 7
    clane = users & 127
    a_t = table_t[:, :NI_PAD]
    buy8 = jnp.broadcast_to(relation_table[-1], (8, DIM))

    rank, pred = _fused(table_t, wcol, clane, items, a_t, buy8)
    return (pred, rank)
